# Initial kernel scaffold; baseline (speedup 1.0000x reference)
import jax
import jax.numpy as jnp
from jax.experimental import pallas as pl


def kernel(x, W):
    raise NotImplementedError("write your pallas kernel here")


# speedup vs baseline: 13.8129x; 13.8129x over previous
import jax
import jax.numpy as jnp
from jax.experimental import pallas as pl


def _bcast_body(w_ref, o_ref):
    o_ref[...] = jnp.broadcast_to(w_ref[...], o_ref.shape)


def kernel(x, W):
    B, S = x.shape
    M, D = W.shape
    rows = W[:S]
    Wf = rows.reshape(1, S * D)
    BB = 256
    out = pl.pallas_call(
        _bcast_body,
        grid=(B // BB,),
        in_specs=[pl.BlockSpec((1, S * D), lambda i: (0, 0))],
        out_specs=pl.BlockSpec((BB, S * D), lambda i: (i, 0)),
        out_shape=jax.ShapeDtypeStruct((B, S * D), jnp.float32),
    )(Wf)
    return out.reshape(B, S, D)


# speedup vs baseline: 13.8148x; 1.0001x over previous
import jax
import jax.numpy as jnp
from jax.experimental import pallas as pl
from jax.experimental.pallas import tpu as pltpu

_R = 64
_Q = 4


def _body(w_ref, o_hbm, scratch, sems):
    n_chunks = o_hbm.shape[0] // _R
    scratch[...] = jnp.broadcast_to(w_ref[...], scratch.shape)

    def loop(i, carry):
        @pl.when(i >= _Q)
        def _():
            pltpu.make_async_copy(
                scratch, o_hbm.at[pl.ds((i - _Q) * _R, _R), :], sems.at[i % _Q]
            ).wait()
        pltpu.make_async_copy(
            scratch, o_hbm.at[pl.ds(i * _R, _R), :], sems.at[i % _Q]
        ).start()
        return carry

    jax.lax.fori_loop(0, n_chunks, loop, 0)
    for q in range(_Q):
        i = n_chunks - _Q + q
        pltpu.make_async_copy(
            scratch, o_hbm.at[pl.ds(i * _R, _R), :], sems.at[i % _Q]
        ).wait()


def kernel(x, W):
    B, S = x.shape
    M, D = W.shape
    ROW = S * D
    Wf = W[:S].reshape(1, ROW)
    out = pl.pallas_call(
        _body,
        in_specs=[pl.BlockSpec(memory_space=pltpu.MemorySpace.VMEM)],
        out_specs=pl.BlockSpec(memory_space=pl.ANY),
        out_shape=jax.ShapeDtypeStruct((B, ROW), jnp.float32),
        scratch_shapes=[
            pltpu.VMEM((_R, ROW), jnp.float32),
            pltpu.SemaphoreType.DMA((_Q,)),
        ],
    )(Wf)
    return out.reshape(B, S, D)
